# baseline (speedup 1.0000x reference)
import functools

import jax
import jax.numpy as jnp
from jax import lax
from jax.experimental import pallas as pl
from jax.experimental.pallas import tpu as pltpu
from jax.experimental.pallas import tpu_sc as plsc

BATCH = 16384
NF = 100
NFIELD = 26
EMB = 16

F_SC = 32
F_TC = NF - F_SC


B_CH = 1024
GRID = BATCH // B_CH


def _tc_body(fi_ref, vt_ref, x_ref, out_ref, et_ref):
    @pl.when(pl.program_id(0) == 0)
    def _build_et():
        c_iota = lax.broadcasted_iota(
            jnp.int32, (NFIELD, NF), 0).astype(jnp.float32)
        onehot = (c_iota == jnp.broadcast_to(fi_ref[...], (NFIELD, NF)))
        et_ref[...] = lax.dot(vt_ref[...], onehot.astype(jnp.float32),
                              preferred_element_type=jnp.float32)

    for f in range(F_SC, NF):
        x_row = x_ref[f:f + 1, :]
        e_col = et_ref[:, f:f + 1]
        out_ref[f - F_SC] = e_col * x_row


def _tc_call(fi_f, v_t, x_t):
    return pl.pallas_call(
        _tc_body,
        grid=(GRID,),
        in_specs=[
            pl.BlockSpec((1, NF), lambda i: (0, 0)),
            pl.BlockSpec((EMB, NFIELD), lambda i: (0, 0)),
            pl.BlockSpec((NF, B_CH), lambda i: (0, i)),
        ],
        out_specs=pl.BlockSpec((F_TC, EMB, B_CH), lambda i: (0, 0, i)),
        out_shape=jax.ShapeDtypeStruct((F_TC, EMB, BATCH), jnp.float32),
        scratch_shapes=[pltpu.VMEM((EMB, NF), jnp.float32)],
        compiler_params=pltpu.CompilerParams(
            dimension_semantics=("arbitrary",),
        ),
    )(fi_f, v_t, x_t)


NC = 2
NS = 16
NW = NC * NS
B_PER_W = BATCH // NW
FB = 8
BB = 256
NCH = (F_SC // FB) * (B_PER_W // BB)

_dnums = lax.GatherDimensionNumbers(
    offset_dims=(), collapsed_slice_dims=(0,), start_index_map=(0,))


def _splat(vec, j):
    idx = jnp.full((16, 1), j, jnp.int32)
    return lax.gather(vec, idx, _dnums, (1,),
                      mode=lax.GatherScatterMode.PROMISE_IN_BOUNDS)


def _sc_body(x_hbm, v_hbm, fi_hbm, out_hbm,
             fi_v, e_v, xall, ob0, ob1,
             esem, os0, os1):
    wid = lax.axis_index("s") * NC + lax.axis_index("c")
    b0 = wid * B_PER_W

    pltpu.sync_copy(fi_hbm, fi_v)
    pltpu.make_async_copy(v_hbm.at[fi_v], e_v, esem).start()
    pltpu.make_async_copy(v_hbm.at[fi_v], e_v, esem).wait()

    pltpu.sync_copy(x_hbm.at[pl.ds(0, F_SC), pl.ds(b0, B_PER_W)], xall)

    obufs = (ob0, ob1)
    osems = (os0, os1)

    def compute(obuf, fc, bl):
        @pl.loop(0, FB)
        def _fl(fl):
            fa = fc * FB + fl
            erow = e_v[fa, :EMB]
            evs = [_splat(erow, e) for e in range(EMB)]
            for k in range(BB // 16):
                xv = xall[fa, pl.ds(bl + k * 16, 16)]
                for e in range(EMB):
                    obuf[fl, e, pl.ds(k * 16, 16)] = evs[e] * xv

    @pl.loop(0, NCH, step=2)
    def _c(c0):
        for s in range(2):
            c = c0 + s
            fc = c // 2
            bl = (c % 2) * BB

            @pl.when(c >= 2)
            def _wait_prev():
                pltpu.make_async_copy(
                    obufs[s],
                    out_hbm.at[pl.ds(0, FB), :, pl.ds(0, BB)],
                    osems[s]).wait()

            compute(obufs[s], fc, bl)
            pltpu.make_async_copy(
                obufs[s],
                out_hbm.at[pl.ds(fc * FB, FB), :, pl.ds(b0 + bl, BB)],
                osems[s]).start()

    for s in range(2):
        pltpu.make_async_copy(
            obufs[s], out_hbm.at[pl.ds(0, FB), :, pl.ds(0, BB)],
            osems[s]).wait()


_sc_call = functools.partial(
    pl.kernel,
    out_type=jax.ShapeDtypeStruct((F_SC, EMB, BATCH), jnp.float32),
    mesh=plsc.VectorSubcoreMesh(core_axis_name="c", subcore_axis_name="s"),
    scratch_types=[
        pltpu.VMEM((NF,), jnp.int32),
        pltpu.VMEM((NF, 128), jnp.float32),
        pltpu.VMEM((F_SC, B_PER_W), jnp.float32),
        pltpu.VMEM((FB, EMB, BB), jnp.float32),
        pltpu.VMEM((FB, EMB, BB), jnp.float32),
        pltpu.SemaphoreType.DMA,
        pltpu.SemaphoreType.DMA,
        pltpu.SemaphoreType.DMA,
    ],
)(_sc_body)


def kernel(inputs, V, field_index):
    x_t = inputs.T
    v_t = V.T
    v_pad = jnp.pad(V, ((0, 0), (0, 128 - EMB)))
    fi_f = field_index.astype(jnp.float32).reshape(1, NF)
    sc_out = _sc_call(x_t, v_pad, field_index)
    tc_out = _tc_call(fi_f, v_t, x_t)
    out_t = jnp.concatenate([sc_out, tc_out], axis=0)
    return jnp.transpose(out_t, (2, 0, 1))


# speedup vs baseline: 3.4286x; 3.4286x over previous
import jax
import jax.numpy as jnp
from jax import lax
from jax.experimental import pallas as pl
from jax.experimental.pallas import tpu as pltpu

BATCH = 16384
NF = 100
NFIELD = 26
EMB = 16
B_CH = 1024
GRID = BATCH // B_CH


def _body(fi_ref, vt_ref, x_ref, out_ref, et_ref):
    @pl.when(pl.program_id(0) == 0)
    def _build_et():
        fi_row = jnp.broadcast_to(fi_ref[...], (EMB, NF))
        et = jnp.zeros((EMB, NF), jnp.float32)
        for c in range(NFIELD):
            et = jnp.where(fi_row == float(c), vt_ref[:, c:c + 1], et)
        et_ref[...] = et

    for f in range(NF):
        x_row = x_ref[f:f + 1, :]
        e_col = et_ref[:, f:f + 1]
        out_ref[f] = e_col * x_row


def kernel(inputs, V, field_index):
    x_t = inputs.T
    v_t = V.T
    fi_f = field_index.astype(jnp.float32).reshape(1, NF)
    out_t = pl.pallas_call(
        _body,
        grid=(GRID,),
        in_specs=[
            pl.BlockSpec((1, NF), lambda i: (0, 0)),
            pl.BlockSpec((EMB, NFIELD), lambda i: (0, 0)),
            pl.BlockSpec((NF, B_CH), lambda i: (0, i)),
        ],
        out_specs=pl.BlockSpec((NF, EMB, B_CH), lambda i: (0, 0, i)),
        out_shape=jax.ShapeDtypeStruct((NF, EMB, BATCH), jnp.float32),
        scratch_shapes=[pltpu.VMEM((EMB, NF), jnp.float32)],
        compiler_params=pltpu.CompilerParams(
            dimension_semantics=("arbitrary",),
        ),
    )(fi_f, v_t, x_t)
    return jnp.transpose(out_t, (2, 0, 1))


# speedup vs baseline: 3.4686x; 1.0117x over previous
import jax
import jax.numpy as jnp
from jax import lax
from jax.experimental import pallas as pl
from jax.experimental.pallas import tpu as pltpu

BATCH = 16384
NF = 100
NFIELD = 26
EMB = 16
B_CH = 2048
GRID = BATCH // B_CH


def _body(fi_ref, vt_ref, x_ref, out_ref, et_ref):
    @pl.when(pl.program_id(0) == 0)
    def _build_et():
        fi_row = jnp.broadcast_to(fi_ref[...], (EMB, NF))
        et = jnp.zeros((EMB, NF), jnp.float32)
        for c in range(NFIELD):
            et = jnp.where(fi_row == float(c), vt_ref[:, c:c + 1], et)
        et_ref[...] = et

    for f in range(NF):
        x_row = x_ref[f:f + 1, :]
        e_col = et_ref[:, f:f + 1]
        out_ref[f] = e_col * x_row


def kernel(inputs, V, field_index):
    x_t = inputs.T
    v_t = V.T
    fi_f = field_index.astype(jnp.float32).reshape(1, NF)
    out_t = pl.pallas_call(
        _body,
        grid=(GRID,),
        in_specs=[
            pl.BlockSpec((1, NF), lambda i: (0, 0)),
            pl.BlockSpec((EMB, NFIELD), lambda i: (0, 0)),
            pl.BlockSpec((NF, B_CH), lambda i: (0, i)),
        ],
        out_specs=pl.BlockSpec((NF, EMB, B_CH), lambda i: (0, 0, i)),
        out_shape=jax.ShapeDtypeStruct((NF, EMB, BATCH), jnp.float32),
        scratch_shapes=[pltpu.VMEM((EMB, NF), jnp.float32)],
        compiler_params=pltpu.CompilerParams(
            dimension_semantics=("arbitrary",),
        ),
    )(fi_f, v_t, x_t)
    return jnp.transpose(out_t, (2, 0, 1))
